# baseline (speedup 1.0000x reference)
import functools

import jax
import jax.numpy as jnp
from jax import lax
from jax.experimental import pallas as pl
from jax.experimental.pallas import tpu as pltpu
from jax.experimental.pallas import tpu_sc as plsc

N_NODES = 10000
IN_DIM = 128
HIDDEN = 256
NUM_CLASSES = 64
N_EDGES = 320000

NC, NS = 2, 16
NW = NC * NS
LANES = 64
D = 128

ROWS_PER_TILE = -(-N_EDGES // (NW * LANES * 8)) * 8
N_ROWS_PAD = ROWS_PER_TILE * NW
E_PAD = N_ROWS_PAD * LANES
N_ACC = N_NODES + LANES

N_PER_TILE = 624
N_REM = N_NODES - N_PER_TILE * NS
ZBUF = N_PER_TILE + N_REM
NBUF = 5
CH = 16

_mesh = plsc.VectorSubcoreMesh(core_axis_name="c", subcore_axis_name="s",
                               num_cores=NC, num_subcores=NS)


@functools.partial(
    pl.kernel,
    out_type=jax.ShapeDtypeStruct((NC, N_NODES, D), jnp.float32),
    mesh=_mesh,
    scratch_types=[
        pltpu.VMEM((CH, LANES), jnp.int32),
        pltpu.VMEM((CH, LANES), jnp.int32),
        pltpu.VMEM((NBUF, LANES, D), jnp.float32),
        pltpu.VMEM_SHARED((N_ACC, D), jnp.float32),
        [pltpu.SemaphoreType.DMA] * NBUF,
    ],
)
def _sc_segsum(table_hbm, src_hbm, dst_hbm, zeros_hbm, out_hbm,
               src_v, dst_v, rows_v, acc_sh, sems):
    c = lax.axis_index("c")
    s = lax.axis_index("s")
    wid = s * NC + c

    pltpu.sync_copy(zeros_hbm.at[pl.ds(0, N_PER_TILE)],
                    acc_sh.at[pl.ds(s * N_PER_TILE, N_PER_TILE)])

    @pl.when(s == 0)
    def _():
        pltpu.sync_copy(zeros_hbm.at[pl.ds(0, N_REM)],
                        acc_sh.at[pl.ds(NS * N_PER_TILE, N_REM)])

    row_base = wid * ROWS_PER_TILE
    plsc.subcore_barrier()

    def _chunk(o, carry):
        pltpu.sync_copy(src_hbm.at[pl.ds(row_base + o * CH, CH)], src_v)
        pltpu.sync_copy(dst_hbm.at[pl.ds(row_base + o * CH, CH)], dst_v)
        for b in range(NBUF):
            pltpu.async_copy(table_hbm.at[src_v.at[b]], rows_v.at[b], sems[b])
        for j in range(CH):
            b = j % NBUF
            pltpu.make_async_copy(table_hbm.at[pl.ds(0, LANES)],
                                  rows_v.at[b], sems[b]).wait()
            pltpu.sync_copy(rows_v.at[b], acc_sh.at[dst_v.at[j]], add=True)
            if j + NBUF < CH:
                pltpu.async_copy(table_hbm.at[src_v.at[j + NBUF]],
                                 rows_v.at[b], sems[b])
        return carry

    lax.fori_loop(0, ROWS_PER_TILE // CH, _chunk, 0)

    plsc.subcore_barrier()

    pltpu.sync_copy(acc_sh.at[pl.ds(s * N_PER_TILE, N_PER_TILE)],
                    out_hbm.at[c, pl.ds(s * N_PER_TILE, N_PER_TILE)])

    @pl.when(s == 0)
    def _():
        pltpu.sync_copy(acc_sh.at[pl.ds(NS * N_PER_TILE, N_REM)],
                        out_hbm.at[c, pl.ds(NS * N_PER_TILE, N_REM)])


BR = 1000


def _mlp1_body(x_ref, p_ref, w1a_ref, b1a_ref, w1b_ref, b1b_ref,
               hlo_ref, hhi_ref):
    hin = x_ref[...] + p_ref[0] + p_ref[1]
    a = jnp.dot(hin, w1a_ref[...], preferred_element_type=jnp.float32)
    a = jnp.maximum(a + b1a_ref[...], 0.0)
    h = jnp.dot(a, w1b_ref[...], preferred_element_type=jnp.float32)
    h = jnp.maximum(h + b1b_ref[...], 0.0)
    hlo_ref[...] = h[:, :D]
    hhi_ref[...] = h[:, D:]


def _mlp2_body(hlo_ref, hhi_ref, plo_ref, phi_ref, w2a_ref, b2a_ref,
               w2b_ref, b2b_ref, out_ref):
    hin = jnp.concatenate(
        [hlo_ref[...] + plo_ref[0] + plo_ref[1],
         hhi_ref[...] + phi_ref[0] + phi_ref[1]], axis=1)
    a = jnp.dot(hin, w2a_ref[...], preferred_element_type=jnp.float32)
    a = jnp.maximum(a + b2a_ref[...], 0.0)
    out_ref[...] = jnp.dot(a, w2b_ref[...], preferred_element_type=jnp.float32) + b2b_ref[...]


def _row_block(d):
    return pl.BlockSpec((BR, d), lambda i: (i, 0))


def _part_block(d):
    return pl.BlockSpec((NC, BR, d), lambda i: (0, i, 0))


def _full(shape):
    return pl.BlockSpec(shape, lambda i: tuple(0 for _ in shape))


_mlp1 = pl.pallas_call(
    _mlp1_body,
    grid=(N_NODES // BR,),
    in_specs=[
        _row_block(IN_DIM),
        _part_block(IN_DIM),
        _full((IN_DIM, HIDDEN)),
        _full((1, HIDDEN)),
        _full((HIDDEN, HIDDEN)),
        _full((1, HIDDEN)),
    ],
    out_specs=[_row_block(D), _row_block(D)],
    out_shape=[
        jax.ShapeDtypeStruct((N_NODES, D), jnp.float32),
        jax.ShapeDtypeStruct((N_NODES, D), jnp.float32),
    ],
)

_mlp2 = pl.pallas_call(
    _mlp2_body,
    grid=(N_NODES // BR,),
    in_specs=[
        _row_block(D),
        _row_block(D),
        _part_block(D),
        _part_block(D),
        _full((HIDDEN, HIDDEN)),
        _full((1, HIDDEN)),
        _full((HIDDEN, NUM_CLASSES)),
        _full((1, NUM_CLASSES)),
    ],
    out_specs=_row_block(NUM_CLASSES),
    out_shape=jax.ShapeDtypeStruct((N_NODES, NUM_CLASSES), jnp.float32),
)


def kernel(x, edge_index, W1a, b1a, W1b, b1b, W2a, b2a, W2b, b2b):
    ei = edge_index.astype(jnp.int32)
    pad = E_PAD - N_EDGES
    lane = jnp.tile(jnp.arange(LANES, dtype=jnp.int32), pad // LANES)
    src = jnp.concatenate([ei[0], lane])
    dst = jnp.concatenate([ei[1], N_NODES + lane])
    src = src.reshape(N_ROWS_PAD, LANES)
    dst = dst.reshape(N_ROWS_PAD, LANES)
    zeros = jnp.zeros((ZBUF, D), jnp.float32)

    p1 = _sc_segsum(x, src, dst, zeros)
    h_lo, h_hi = _mlp1(x, p1, W1a, b1a.reshape(1, -1), W1b, b1b.reshape(1, -1))
    p2_lo = _sc_segsum(h_lo, src, dst, zeros)
    p2_hi = _sc_segsum(h_hi, src, dst, zeros)
    return _mlp2(h_lo, h_hi, p2_lo, p2_hi, W2a, b2a.reshape(1, -1),
                 W2b, b2b.reshape(1, -1))


# speedup vs baseline: 1.1027x; 1.1027x over previous
import functools

import jax
import jax.numpy as jnp
from jax import lax
from jax.experimental import pallas as pl
from jax.experimental.pallas import tpu as pltpu
from jax.experimental.pallas import tpu_sc as plsc

N_NODES = 10000
IN_DIM = 128
HIDDEN = 256
NUM_CLASSES = 64
N_EDGES = 320000

NC, NS = 2, 16
NW = NC * NS
LANES = 128
D = 128

ROWS_PER_TILE = -(-N_EDGES // (NW * LANES * 8)) * 8
N_ROWS_PAD = ROWS_PER_TILE * NW
E_PAD = N_ROWS_PAD * LANES
N_ACC = N_NODES + LANES

N_PER_TILE = 624
N_REM = N_NODES - N_PER_TILE * NS
ZBUF = N_PER_TILE + N_REM
NBUF = 2
CH = 40

_mesh = plsc.VectorSubcoreMesh(core_axis_name="c", subcore_axis_name="s",
                               num_cores=NC, num_subcores=NS)


@functools.partial(
    pl.kernel,
    out_type=jax.ShapeDtypeStruct((NC, N_NODES, D), jnp.float32),
    mesh=_mesh,
    scratch_types=[
        pltpu.VMEM((CH, LANES), jnp.int32),
        pltpu.VMEM((CH, LANES), jnp.int32),
        pltpu.VMEM((NBUF, LANES, D), jnp.float32),
        pltpu.VMEM_SHARED((N_ACC, D), jnp.float32),
        [pltpu.SemaphoreType.DMA] * NBUF,
    ],
)
def _sc_segsum(table_hbm, src_hbm, dst_hbm, zeros_hbm, out_hbm,
               src_v, dst_v, rows_v, acc_sh, sems):
    c = lax.axis_index("c")
    s = lax.axis_index("s")
    wid = s * NC + c

    pltpu.sync_copy(zeros_hbm.at[pl.ds(0, N_PER_TILE)],
                    acc_sh.at[pl.ds(s * N_PER_TILE, N_PER_TILE)])

    @pl.when(s == 0)
    def _():
        pltpu.sync_copy(zeros_hbm.at[pl.ds(0, N_REM)],
                        acc_sh.at[pl.ds(NS * N_PER_TILE, N_REM)])

    row_base = wid * ROWS_PER_TILE
    plsc.subcore_barrier()

    def _chunk(o, carry):
        pltpu.sync_copy(src_hbm.at[pl.ds(row_base + o * CH, CH)], src_v)
        pltpu.sync_copy(dst_hbm.at[pl.ds(row_base + o * CH, CH)], dst_v)
        for b in range(NBUF):
            pltpu.async_copy(table_hbm.at[src_v.at[b]], rows_v.at[b], sems[b])
        for j in range(CH):
            b = j % NBUF
            pltpu.make_async_copy(table_hbm.at[pl.ds(0, LANES)],
                                  rows_v.at[b], sems[b]).wait()
            pltpu.sync_copy(rows_v.at[b], acc_sh.at[dst_v.at[j]], add=True)
            if j + NBUF < CH:
                pltpu.async_copy(table_hbm.at[src_v.at[j + NBUF]],
                                 rows_v.at[b], sems[b])
        return carry

    lax.fori_loop(0, ROWS_PER_TILE // CH, _chunk, 0)

    plsc.subcore_barrier()

    pltpu.sync_copy(acc_sh.at[pl.ds(s * N_PER_TILE, N_PER_TILE)],
                    out_hbm.at[c, pl.ds(s * N_PER_TILE, N_PER_TILE)])

    @pl.when(s == 0)
    def _():
        pltpu.sync_copy(acc_sh.at[pl.ds(NS * N_PER_TILE, N_REM)],
                        out_hbm.at[c, pl.ds(NS * N_PER_TILE, N_REM)])


BR = 1000


def _mlp1_body(x_ref, p_ref, w1a_ref, b1a_ref, w1b_ref, b1b_ref,
               hlo_ref, hhi_ref):
    hin = x_ref[...] + p_ref[0] + p_ref[1]
    a = jnp.dot(hin, w1a_ref[...], preferred_element_type=jnp.float32)
    a = jnp.maximum(a + b1a_ref[...], 0.0)
    h = jnp.dot(a, w1b_ref[...], preferred_element_type=jnp.float32)
    h = jnp.maximum(h + b1b_ref[...], 0.0)
    hlo_ref[...] = h[:, :D]
    hhi_ref[...] = h[:, D:]


def _mlp2_body(hlo_ref, hhi_ref, plo_ref, phi_ref, w2a_ref, b2a_ref,
               w2b_ref, b2b_ref, out_ref):
    hin = jnp.concatenate(
        [hlo_ref[...] + plo_ref[0] + plo_ref[1],
         hhi_ref[...] + phi_ref[0] + phi_ref[1]], axis=1)
    a = jnp.dot(hin, w2a_ref[...], preferred_element_type=jnp.float32)
    a = jnp.maximum(a + b2a_ref[...], 0.0)
    out_ref[...] = jnp.dot(a, w2b_ref[...], preferred_element_type=jnp.float32) + b2b_ref[...]


def _row_block(d):
    return pl.BlockSpec((BR, d), lambda i: (i, 0))


def _part_block(d):
    return pl.BlockSpec((NC, BR, d), lambda i: (0, i, 0))


def _full(shape):
    return pl.BlockSpec(shape, lambda i: tuple(0 for _ in shape))


_mlp1 = pl.pallas_call(
    _mlp1_body,
    grid=(N_NODES // BR,),
    in_specs=[
        _row_block(IN_DIM),
        _part_block(IN_DIM),
        _full((IN_DIM, HIDDEN)),
        _full((1, HIDDEN)),
        _full((HIDDEN, HIDDEN)),
        _full((1, HIDDEN)),
    ],
    out_specs=[_row_block(D), _row_block(D)],
    out_shape=[
        jax.ShapeDtypeStruct((N_NODES, D), jnp.float32),
        jax.ShapeDtypeStruct((N_NODES, D), jnp.float32),
    ],
)

_mlp2 = pl.pallas_call(
    _mlp2_body,
    grid=(N_NODES // BR,),
    in_specs=[
        _row_block(D),
        _row_block(D),
        _part_block(D),
        _part_block(D),
        _full((HIDDEN, HIDDEN)),
        _full((1, HIDDEN)),
        _full((HIDDEN, NUM_CLASSES)),
        _full((1, NUM_CLASSES)),
    ],
    out_specs=_row_block(NUM_CLASSES),
    out_shape=jax.ShapeDtypeStruct((N_NODES, NUM_CLASSES), jnp.float32),
)


def kernel(x, edge_index, W1a, b1a, W1b, b1b, W2a, b2a, W2b, b2b):
    ei = edge_index.astype(jnp.int32)
    pad = E_PAD - N_EDGES
    lane = jnp.tile(jnp.arange(LANES, dtype=jnp.int32), pad // LANES)
    src = jnp.concatenate([ei[0], lane])
    dst = jnp.concatenate([ei[1], N_NODES + lane])
    src = src.reshape(N_ROWS_PAD, LANES)
    dst = dst.reshape(N_ROWS_PAD, LANES)
    zeros = jnp.zeros((ZBUF, D), jnp.float32)

    p1 = _sc_segsum(x, src, dst, zeros)
    h_lo, h_hi = _mlp1(x, p1, W1a, b1a.reshape(1, -1), W1b, b1b.reshape(1, -1))
    p2_lo = _sc_segsum(h_lo, src, dst, zeros)
    p2_hi = _sc_segsum(h_hi, src, dst, zeros)
    return _mlp2(h_lo, h_hi, p2_lo, p2_hi, W2a, b2a.reshape(1, -1),
                 W2b, b2b.reshape(1, -1))


# speedup vs baseline: 1.1049x; 1.0020x over previous
import functools

import jax
import jax.numpy as jnp
from jax import lax
from jax.experimental import pallas as pl
from jax.experimental.pallas import tpu as pltpu
from jax.experimental.pallas import tpu_sc as plsc

N_NODES = 10000
IN_DIM = 128
HIDDEN = 256
NUM_CLASSES = 64
N_EDGES = 320000

NC, NS = 2, 16
NW = NC * NS
LANES = 128
D = 128

N_ROWS = N_EDGES // LANES
ROWS_PER_TILE = 80
TAIL_ROWS = N_ROWS - 31 * ROWS_PER_TILE
N_ACC = N_NODES

N_PER_TILE = 624
N_REM = N_NODES - N_PER_TILE * NS
ZBUF = N_PER_TILE + N_REM
NBUF = 2
CH = 40

_mesh = plsc.VectorSubcoreMesh(core_axis_name="c", subcore_axis_name="s",
                               num_cores=NC, num_subcores=NS)


@functools.partial(
    pl.kernel,
    out_type=jax.ShapeDtypeStruct((NC, N_NODES, D), jnp.float32),
    mesh=_mesh,
    scratch_types=[
        pltpu.VMEM((CH, LANES), jnp.int32),
        pltpu.VMEM((CH, LANES), jnp.int32),
        pltpu.VMEM((NBUF, LANES, D), jnp.float32),
        pltpu.VMEM_SHARED((N_ACC, D), jnp.float32),
        [pltpu.SemaphoreType.DMA] * NBUF,
    ],
)
def _sc_segsum(table_hbm, src_hbm, dst_hbm, zeros_hbm, out_hbm,
               src_v, dst_v, rows_v, acc_sh, sems):
    c = lax.axis_index("c")
    s = lax.axis_index("s")
    wid = s * NC + c

    pltpu.sync_copy(zeros_hbm.at[pl.ds(0, N_PER_TILE)],
                    acc_sh.at[pl.ds(s * N_PER_TILE, N_PER_TILE)])

    @pl.when(s == 0)
    def _():
        pltpu.sync_copy(zeros_hbm.at[pl.ds(0, N_REM)],
                        acc_sh.at[pl.ds(NS * N_PER_TILE, N_REM)])

    row_base = wid * ROWS_PER_TILE
    plsc.subcore_barrier()

    def _run_chunk(base, nrows):
        pltpu.sync_copy(src_hbm.at[pl.ds(base, nrows)],
                        src_v.at[pl.ds(0, nrows)])
        pltpu.sync_copy(dst_hbm.at[pl.ds(base, nrows)],
                        dst_v.at[pl.ds(0, nrows)])
        for b in range(min(NBUF, nrows)):
            pltpu.async_copy(table_hbm.at[src_v.at[b]], rows_v.at[b], sems[b])
        for j in range(nrows):
            b = j % NBUF
            pltpu.make_async_copy(table_hbm.at[pl.ds(0, LANES)],
                                  rows_v.at[b], sems[b]).wait()
            pltpu.sync_copy(rows_v.at[b], acc_sh.at[dst_v.at[j]], add=True)
            if j + NBUF < nrows:
                pltpu.async_copy(table_hbm.at[src_v.at[j + NBUF]],
                                 rows_v.at[b], sems[b])

    @pl.when(wid < NW - 1)
    def _():
        def _chunk(o, carry):
            _run_chunk(row_base + o * CH, CH)
            return carry
        lax.fori_loop(0, ROWS_PER_TILE // CH, _chunk, 0)

    @pl.when(wid == NW - 1)
    def _():
        _run_chunk((NW - 1) * ROWS_PER_TILE, TAIL_ROWS)

    plsc.subcore_barrier()

    pltpu.sync_copy(acc_sh.at[pl.ds(s * N_PER_TILE, N_PER_TILE)],
                    out_hbm.at[c, pl.ds(s * N_PER_TILE, N_PER_TILE)])

    @pl.when(s == 0)
    def _():
        pltpu.sync_copy(acc_sh.at[pl.ds(NS * N_PER_TILE, N_REM)],
                        out_hbm.at[c, pl.ds(NS * N_PER_TILE, N_REM)])


BR = 1000


def _mlp1_body(x_ref, p_ref, w1a_ref, b1a_ref, w1b_ref, b1b_ref,
               hlo_ref, hhi_ref):
    hin = x_ref[...] + p_ref[0] + p_ref[1]
    a = jnp.dot(hin, w1a_ref[...], preferred_element_type=jnp.float32)
    a = jnp.maximum(a + b1a_ref[...], 0.0)
    h = jnp.dot(a, w1b_ref[...], preferred_element_type=jnp.float32)
    h = jnp.maximum(h + b1b_ref[...], 0.0)
    hlo_ref[...] = h[:, :D]
    hhi_ref[...] = h[:, D:]


def _mlp2_body(hlo_ref, hhi_ref, plo_ref, phi_ref, w2a_ref, b2a_ref,
               w2b_ref, b2b_ref, out_ref):
    hin = jnp.concatenate(
        [hlo_ref[...] + plo_ref[0] + plo_ref[1],
         hhi_ref[...] + phi_ref[0] + phi_ref[1]], axis=1)
    a = jnp.dot(hin, w2a_ref[...], preferred_element_type=jnp.float32)
    a = jnp.maximum(a + b2a_ref[...], 0.0)
    out_ref[...] = jnp.dot(a, w2b_ref[...], preferred_element_type=jnp.float32) + b2b_ref[...]


def _row_block(d):
    return pl.BlockSpec((BR, d), lambda i: (i, 0))


def _part_block(d):
    return pl.BlockSpec((NC, BR, d), lambda i: (0, i, 0))


def _full(shape):
    return pl.BlockSpec(shape, lambda i: tuple(0 for _ in shape))


_mlp1 = pl.pallas_call(
    _mlp1_body,
    grid=(N_NODES // BR,),
    in_specs=[
        _row_block(IN_DIM),
        _part_block(IN_DIM),
        _full((IN_DIM, HIDDEN)),
        _full((1, HIDDEN)),
        _full((HIDDEN, HIDDEN)),
        _full((1, HIDDEN)),
    ],
    out_specs=[_row_block(D), _row_block(D)],
    out_shape=[
        jax.ShapeDtypeStruct((N_NODES, D), jnp.float32),
        jax.ShapeDtypeStruct((N_NODES, D), jnp.float32),
    ],
)

_mlp2 = pl.pallas_call(
    _mlp2_body,
    grid=(N_NODES // BR,),
    in_specs=[
        _row_block(D),
        _row_block(D),
        _part_block(D),
        _part_block(D),
        _full((HIDDEN, HIDDEN)),
        _full((1, HIDDEN)),
        _full((HIDDEN, NUM_CLASSES)),
        _full((1, NUM_CLASSES)),
    ],
    out_specs=_row_block(NUM_CLASSES),
    out_shape=jax.ShapeDtypeStruct((N_NODES, NUM_CLASSES), jnp.float32),
)


def kernel(x, edge_index, W1a, b1a, W1b, b1b, W2a, b2a, W2b, b2b):
    ei = edge_index.astype(jnp.int32)
    src = ei[0].reshape(N_ROWS, LANES)
    dst = ei[1].reshape(N_ROWS, LANES)
    zeros = jnp.zeros((ZBUF, D), jnp.float32)

    p1 = _sc_segsum(x, src, dst, zeros)
    h_lo, h_hi = _mlp1(x, p1, W1a, b1a.reshape(1, -1), W1b, b1b.reshape(1, -1))
    p2_lo = _sc_segsum(h_lo, src, dst, zeros)
    p2_hi = _sc_segsum(h_hi, src, dst, zeros)
    return _mlp2(h_lo, h_hi, p2_lo, p2_hi, W2a, b2a.reshape(1, -1),
                 W2b, b2b.reshape(1, -1))


# speedup vs baseline: 1.1422x; 1.0337x over previous
import functools

import jax
import jax.numpy as jnp
from jax import lax
from jax.experimental import pallas as pl
from jax.experimental.pallas import tpu as pltpu
from jax.experimental.pallas import tpu_sc as plsc

N_NODES = 10000
IN_DIM = 128
HIDDEN = 256
NUM_CLASSES = 64
N_EDGES = 320000

NC, NS = 2, 16
NW = NC * NS
LANES = 128
D = 128

N_ROWS = N_EDGES // LANES
ROWS_PER_TILE = 80
TAIL_ROWS = N_ROWS - 31 * ROWS_PER_TILE
N_ACC = N_NODES

N_PER_TILE = 624
N_REM = N_NODES - N_PER_TILE * NS
ZBUF = N_PER_TILE + N_REM
NBUF = 2
CH = 40

_mesh = plsc.VectorSubcoreMesh(core_axis_name="c", subcore_axis_name="s",
                               num_cores=NC, num_subcores=NS)


@functools.partial(
    pl.kernel,
    out_type=jax.ShapeDtypeStruct((NC, N_NODES, D), jnp.float32),
    mesh=_mesh,
    scratch_types=[
        pltpu.VMEM((CH, LANES), jnp.int32),
        pltpu.VMEM((CH, LANES), jnp.int32),
        pltpu.VMEM((NBUF, LANES, D), jnp.float32),
        pltpu.VMEM_SHARED((N_ACC, D), jnp.float32),
        [pltpu.SemaphoreType.DMA] * NBUF,
    ],
)
def _sc_segsum(table_hbm, edge_hbm, zeros_hbm, out_hbm,
               src_v, dst_v, rows_v, acc_sh, sems):
    c = lax.axis_index("c")
    s = lax.axis_index("s")
    wid = s * NC + c

    pltpu.sync_copy(zeros_hbm.at[pl.ds(0, N_PER_TILE)],
                    acc_sh.at[pl.ds(s * N_PER_TILE, N_PER_TILE)])

    @pl.when(s == 0)
    def _():
        pltpu.sync_copy(zeros_hbm.at[pl.ds(0, N_REM)],
                        acc_sh.at[pl.ds(NS * N_PER_TILE, N_REM)])

    row_base = wid * ROWS_PER_TILE
    plsc.subcore_barrier()

    def _run_chunk(base, nrows):
        pltpu.sync_copy(edge_hbm.at[0, pl.ds(base, nrows)],
                        src_v.at[pl.ds(0, nrows)])
        pltpu.sync_copy(edge_hbm.at[1, pl.ds(base, nrows)],
                        dst_v.at[pl.ds(0, nrows)])
        for b in range(min(NBUF, nrows)):
            pltpu.async_copy(table_hbm.at[src_v.at[b]], rows_v.at[b], sems[b])
        for j in range(nrows):
            b = j % NBUF
            pltpu.make_async_copy(table_hbm.at[pl.ds(0, LANES)],
                                  rows_v.at[b], sems[b]).wait()
            pltpu.sync_copy(rows_v.at[b], acc_sh.at[dst_v.at[j]], add=True)
            if j + NBUF < nrows:
                pltpu.async_copy(table_hbm.at[src_v.at[j + NBUF]],
                                 rows_v.at[b], sems[b])

    @pl.when(wid < NW - 1)
    def _():
        def _chunk(o, carry):
            _run_chunk(row_base + o * CH, CH)
            return carry
        lax.fori_loop(0, ROWS_PER_TILE // CH, _chunk, 0)

    @pl.when(wid == NW - 1)
    def _():
        _run_chunk((NW - 1) * ROWS_PER_TILE, TAIL_ROWS)

    plsc.subcore_barrier()

    pltpu.sync_copy(acc_sh.at[pl.ds(s * N_PER_TILE, N_PER_TILE)],
                    out_hbm.at[c, pl.ds(s * N_PER_TILE, N_PER_TILE)])

    @pl.when(s == 0)
    def _():
        pltpu.sync_copy(acc_sh.at[pl.ds(NS * N_PER_TILE, N_REM)],
                        out_hbm.at[c, pl.ds(NS * N_PER_TILE, N_REM)])


BR = 1000


def _mlp1_body(x_ref, p_ref, w1a_ref, b1a_ref, w1b_ref, b1b_ref,
               hlo_ref, hhi_ref):
    hin = x_ref[...] + p_ref[0] + p_ref[1]
    a = jnp.dot(hin, w1a_ref[...], preferred_element_type=jnp.float32)
    a = jnp.maximum(a + b1a_ref[...], 0.0)
    h = jnp.dot(a, w1b_ref[...], preferred_element_type=jnp.float32)
    h = jnp.maximum(h + b1b_ref[...], 0.0)
    hlo_ref[...] = h[:, :D]
    hhi_ref[...] = h[:, D:]


def _mlp2_body(hlo_ref, hhi_ref, plo_ref, phi_ref, w2a_ref, b2a_ref,
               w2b_ref, b2b_ref, out_ref):
    hin = jnp.concatenate(
        [hlo_ref[...] + plo_ref[0] + plo_ref[1],
         hhi_ref[...] + phi_ref[0] + phi_ref[1]], axis=1)
    a = jnp.dot(hin, w2a_ref[...], preferred_element_type=jnp.float32)
    a = jnp.maximum(a + b2a_ref[...], 0.0)
    out_ref[...] = jnp.dot(a, w2b_ref[...], preferred_element_type=jnp.float32) + b2b_ref[...]


def _row_block(d):
    return pl.BlockSpec((BR, d), lambda i: (i, 0))


def _part_block(d):
    return pl.BlockSpec((NC, BR, d), lambda i: (0, i, 0))


def _full(shape):
    return pl.BlockSpec(shape, lambda i: tuple(0 for _ in shape))


_mlp1 = pl.pallas_call(
    _mlp1_body,
    grid=(N_NODES // BR,),
    in_specs=[
        _row_block(IN_DIM),
        _part_block(IN_DIM),
        _full((IN_DIM, HIDDEN)),
        _full((1, HIDDEN)),
        _full((HIDDEN, HIDDEN)),
        _full((1, HIDDEN)),
    ],
    out_specs=[_row_block(D), _row_block(D)],
    out_shape=[
        jax.ShapeDtypeStruct((N_NODES, D), jnp.float32),
        jax.ShapeDtypeStruct((N_NODES, D), jnp.float32),
    ],
)

_mlp2 = pl.pallas_call(
    _mlp2_body,
    grid=(N_NODES // BR,),
    in_specs=[
        _row_block(D),
        _row_block(D),
        _part_block(D),
        _part_block(D),
        _full((HIDDEN, HIDDEN)),
        _full((1, HIDDEN)),
        _full((HIDDEN, NUM_CLASSES)),
        _full((1, NUM_CLASSES)),
    ],
    out_specs=_row_block(NUM_CLASSES),
    out_shape=jax.ShapeDtypeStruct((N_NODES, NUM_CLASSES), jnp.float32),
)


def kernel(x, edge_index, W1a, b1a, W1b, b1b, W2a, b2a, W2b, b2b):
    ei3 = edge_index.astype(jnp.int32).reshape(2, N_ROWS, LANES)
    zeros = jnp.zeros((ZBUF, D), jnp.float32)

    p1 = _sc_segsum(x, ei3, zeros)
    h_lo, h_hi = _mlp1(x, p1, W1a, b1a.reshape(1, -1), W1b, b1b.reshape(1, -1))
    p2_lo = _sc_segsum(h_lo, ei3, zeros)
    p2_hi = _sc_segsum(h_hi, ei3, zeros)
    return _mlp2(h_lo, h_hi, p2_lo, p2_hi, W2a, b2a.reshape(1, -1),
                 W2b, b2b.reshape(1, -1))


# speedup vs baseline: 1.1618x; 1.0172x over previous
import functools

import jax
import jax.numpy as jnp
from jax import lax
from jax.experimental import pallas as pl
from jax.experimental.pallas import tpu as pltpu
from jax.experimental.pallas import tpu_sc as plsc

N_NODES = 10000
IN_DIM = 128
HIDDEN = 256
NUM_CLASSES = 64
N_EDGES = 320000

NC, NS = 2, 16
NW = NC * NS
LANES = 128
D = 128

N_ROWS = N_EDGES // LANES
ROWS_PER_TILE = 80
TAIL_ROWS = N_ROWS - 31 * ROWS_PER_TILE
N_ACC = N_NODES

N_PER_TILE = 624
N_REM = N_NODES - N_PER_TILE * NS
ZBUF = N_PER_TILE + N_REM
NBUF = 2
CH = 40

_mesh = plsc.VectorSubcoreMesh(core_axis_name="c", subcore_axis_name="s",
                               num_cores=NC, num_subcores=NS)


def _segsum_body(table_hbm, edge_hbm, zeros_hbm, out_hbm,
                 src_v, dst_v, rows_v, acc_sh, sems):
    c = lax.axis_index("c")
    s = lax.axis_index("s")
    wid = s * NC + c

    pltpu.sync_copy(zeros_hbm.at[pl.ds(0, N_PER_TILE)],
                    acc_sh.at[pl.ds(s * N_PER_TILE, N_PER_TILE)])

    @pl.when(s == 0)
    def _():
        pltpu.sync_copy(zeros_hbm.at[pl.ds(0, N_REM)],
                        acc_sh.at[pl.ds(NS * N_PER_TILE, N_REM)])

    row_base = wid * ROWS_PER_TILE
    plsc.subcore_barrier()

    def _run_chunk(base, nrows):
        pltpu.sync_copy(edge_hbm.at[0, pl.ds(base, nrows)],
                        src_v.at[pl.ds(0, nrows)])
        pltpu.sync_copy(edge_hbm.at[1, pl.ds(base, nrows)],
                        dst_v.at[pl.ds(0, nrows)])
        for b in range(min(NBUF, nrows)):
            pltpu.async_copy(table_hbm.at[src_v.at[b]], rows_v.at[b], sems[b])
        for j in range(nrows):
            b = j % NBUF
            pltpu.make_async_copy(table_hbm.at[pl.ds(0, LANES)],
                                  rows_v.at[b], sems[b]).wait()
            pltpu.sync_copy(rows_v.at[b], acc_sh.at[dst_v.at[j]], add=True)
            if j + NBUF < nrows:
                pltpu.async_copy(table_hbm.at[src_v.at[j + NBUF]],
                                 rows_v.at[b], sems[b])

    @pl.when(wid < NW - 1)
    def _():
        def _chunk(o, carry):
            _run_chunk(row_base + o * CH, CH)
            return carry
        lax.fori_loop(0, ROWS_PER_TILE // CH, _chunk, 0)

    @pl.when(wid == NW - 1)
    def _():
        _run_chunk((NW - 1) * ROWS_PER_TILE, TAIL_ROWS)

    plsc.subcore_barrier()

    pltpu.sync_copy(acc_sh.at[pl.ds(s * N_PER_TILE, N_PER_TILE)],
                    out_hbm.at[c, pl.ds(s * N_PER_TILE, N_PER_TILE)])

    @pl.when(s == 0)
    def _():
        pltpu.sync_copy(acc_sh.at[pl.ds(NS * N_PER_TILE, N_REM)],
                        out_hbm.at[c, pl.ds(NS * N_PER_TILE, N_REM)])


_SC_SCRATCH = [
    pltpu.VMEM((CH, LANES), jnp.int32),
    pltpu.VMEM((CH, LANES), jnp.int32),
    pltpu.VMEM((NBUF, LANES, D), jnp.float32),
    pltpu.VMEM_SHARED((N_ACC, D), jnp.float32),
    [pltpu.SemaphoreType.DMA] * NBUF,
]

_PART = jax.ShapeDtypeStruct((NC, N_NODES, D), jnp.float32)


@functools.partial(pl.kernel, out_type=_PART, mesh=_mesh,
                   scratch_types=_SC_SCRATCH)
def _sc_segsum(table_hbm, edge_hbm, zeros_hbm, out_hbm,
               src_v, dst_v, rows_v, acc_sh, sems):
    _segsum_body(table_hbm, edge_hbm, zeros_hbm, out_hbm,
                 src_v, dst_v, rows_v, acc_sh, sems)


@functools.partial(pl.kernel, out_type=[_PART, _PART], mesh=_mesh,
                   scratch_types=_SC_SCRATCH)
def _sc_segsum2(tlo_hbm, thi_hbm, edge_hbm, zeros_hbm, olo_hbm, ohi_hbm,
                src_v, dst_v, rows_v, acc_sh, sems):
    _segsum_body(tlo_hbm, edge_hbm, zeros_hbm, olo_hbm,
                 src_v, dst_v, rows_v, acc_sh, sems)
    _segsum_body(thi_hbm, edge_hbm, zeros_hbm, ohi_hbm,
                 src_v, dst_v, rows_v, acc_sh, sems)


BR = 1000


def _mlp1_body(x_ref, p_ref, w1a_ref, b1a_ref, w1b_ref, b1b_ref,
               hlo_ref, hhi_ref):
    hin = x_ref[...] + p_ref[0] + p_ref[1]
    a = jnp.dot(hin, w1a_ref[...], preferred_element_type=jnp.float32)
    a = jnp.maximum(a + b1a_ref[...], 0.0)
    h = jnp.dot(a, w1b_ref[...], preferred_element_type=jnp.float32)
    h = jnp.maximum(h + b1b_ref[...], 0.0)
    hlo_ref[...] = h[:, :D]
    hhi_ref[...] = h[:, D:]


def _mlp2_body(hlo_ref, hhi_ref, plo_ref, phi_ref, w2a_ref, b2a_ref,
               w2b_ref, b2b_ref, out_ref):
    hin = jnp.concatenate(
        [hlo_ref[...] + plo_ref[0] + plo_ref[1],
         hhi_ref[...] + phi_ref[0] + phi_ref[1]], axis=1)
    a = jnp.dot(hin, w2a_ref[...], preferred_element_type=jnp.float32)
    a = jnp.maximum(a + b2a_ref[...], 0.0)
    out_ref[...] = jnp.dot(a, w2b_ref[...], preferred_element_type=jnp.float32) + b2b_ref[...]


def _row_block(d):
    return pl.BlockSpec((BR, d), lambda i: (i, 0))


def _part_block(d):
    return pl.BlockSpec((NC, BR, d), lambda i: (0, i, 0))


def _full(shape):
    return pl.BlockSpec(shape, lambda i: tuple(0 for _ in shape))


_mlp1 = pl.pallas_call(
    _mlp1_body,
    grid=(N_NODES // BR,),
    in_specs=[
        _row_block(IN_DIM),
        _part_block(IN_DIM),
        _full((IN_DIM, HIDDEN)),
        _full((1, HIDDEN)),
        _full((HIDDEN, HIDDEN)),
        _full((1, HIDDEN)),
    ],
    out_specs=[_row_block(D), _row_block(D)],
    out_shape=[
        jax.ShapeDtypeStruct((N_NODES, D), jnp.float32),
        jax.ShapeDtypeStruct((N_NODES, D), jnp.float32),
    ],
)

_mlp2 = pl.pallas_call(
    _mlp2_body,
    grid=(N_NODES // BR,),
    in_specs=[
        _row_block(D),
        _row_block(D),
        _part_block(D),
        _part_block(D),
        _full((HIDDEN, HIDDEN)),
        _full((1, HIDDEN)),
        _full((HIDDEN, NUM_CLASSES)),
        _full((1, NUM_CLASSES)),
    ],
    out_specs=_row_block(NUM_CLASSES),
    out_shape=jax.ShapeDtypeStruct((N_NODES, NUM_CLASSES), jnp.float32),
)


def kernel(x, edge_index, W1a, b1a, W1b, b1b, W2a, b2a, W2b, b2b):
    ei3 = edge_index.astype(jnp.int32).reshape(2, N_ROWS, LANES)
    zeros = jnp.zeros((ZBUF, D), jnp.float32)

    p1 = _sc_segsum(x, ei3, zeros)
    h_lo, h_hi = _mlp1(x, p1, W1a, b1a.reshape(1, -1), W1b, b1b.reshape(1, -1))
    p2_lo, p2_hi = _sc_segsum2(h_lo, h_hi, ei3, zeros)
    return _mlp2(h_lo, h_hi, p2_lo, p2_hi, W2a, b2a.reshape(1, -1),
                 W2b, b2b.reshape(1, -1))
